# baseline (speedup 1.0000x reference)
import functools

import jax
import jax.numpy as jnp
from jax import lax
from jax.experimental import pallas as pl
from jax.experimental.pallas import tpu as pltpu
from jax.experimental.pallas import tpu_sc as plsc


@functools.partial(jax.jit, static_argnames=("n", "h", "w", "c_chunk"))
def _run(u1, v1, z1, jraw, pats, *, n, h, w, c_chunk):
    NC, NS = 2, 16
    NW = NC * NS
    ppw = n // NW
    nchunks = ppw // c_chunk
    C = c_chunk
    G = C // 128
    ntx = (w + 127) // 128
    planewords = ((h + 7) // 8) * ntx * 1024

    mesh = plsc.VectorSubcoreMesh(core_axis_name="c", subcore_axis_name="s")

    per_par = [
        pltpu.VMEM((C,), jnp.int32),
        pltpu.VMEM((C,), jnp.int32),
        pltpu.VMEM((3 * C,), jnp.int32),
        pltpu.VMEM((3 * C,), jnp.float32),
        pltpu.VMEM((C // 128, 4, 128), jnp.float32),
    ]
    zbufs = [pltpu.VMEM((C,), jnp.float32)] * 4

    @functools.partial(
        pl.kernel,
        mesh=mesh,
        out_type=jax.ShapeDtypeStruct((n // 128, 4, 128), jnp.float32),
        scratch_types=per_par + per_par + zbufs + [
            pltpu.VMEM((9, 16), jnp.float32),
            pltpu.SemaphoreType.DMA,
            pltpu.SemaphoreType.DMA,
            pltpu.SemaphoreType.DMA,
            pltpu.SemaphoreType.DMA,
            pltpu.SemaphoreType.DMA,
            pltpu.SemaphoreType.DMA,
        ],
    )
    def kern(u_hbm, v_hbm, z_hbm, j_hbm, p_hbm, out_hbm,
             ua, va, ia, ga, oa, ub, vb, ib, gb, ob,
             zq0, zq1, zq2, zq3,
             patbuf, sg0, sg1, si0, si1, so0, so1):
        wid = lax.axis_index("s") * NC + lax.axis_index("c")
        base = wid * ppw
        pltpu.sync_copy(p_hbm, patbuf)

        par_sets = ((ua, va, ia, ga, oa, sg0, si0, so0),
                    (ub, vb, ib, gb, ob, sg1, si1, so1))
        zsets = (zq0, zq1, zq2, zq3)

        def load_pats():
            return tuple(patbuf[r, :] for r in range(9))

        def idx_step(gg, b, uref, vref, iref):
            uu = uref[pl.ds(gg * 128 + b * 16, 16)]
            vv = vref[pl.ds(gg * 128 + b * 16, 16)]
            ty = lax.shift_right_logical(vv, 3)
            ry = lax.bitwise_and(vv, 7)
            tx = lax.shift_right_logical(uu, 7)
            rx = lax.bitwise_and(uu, 127)
            base_idx = (ty * ntx + tx) * 1024 + ry * 128 + rx
            dst0 = gg * 384 + b * 16
            iref[pl.ds(dst0, 16)] = base_idx
            iref[pl.ds(dst0 + 128, 16)] = base_idx + planewords
            iref[pl.ds(dst0 + 256, 16)] = base_idx + 2 * planewords

        def model_step(gg, b, zref, gref, oref, pv):
            zz = zref[pl.ds(gg * 128 + b * 16, 16)]
            src0 = gg * 384 + b * 16
            for c in range(3):
                gval = gref[pl.ds(src0 + c * 128, 16)]
                e1 = jnp.exp(zz * pv[c])
                e2 = jnp.exp(zz * pv[3 + c])
                oref[gg, c, pl.ds(b * 16, 16)] = (
                    gval * e1 + pv[6 + c] * (1.0 - e2))

        def launch(k, par):
            uref, vref, iref, gref, _, sg, si, so = par_sets[par & 1]
            zref = zsets[par & 3]
            p0 = base + k * C
            pltpu.sync_copy(u_hbm.at[pl.ds(p0, C)], uref)
            pltpu.sync_copy(v_hbm.at[pl.ds(p0, C)], vref)
            pltpu.sync_copy(z_hbm.at[pl.ds(p0, C)], zref)

            def idx_only(gg, carry):
                for b in range(8):
                    idx_step(gg, b, uref, vref, iref)
                return carry

            lax.fori_loop(0, G, idx_only, 0)
            pltpu.async_copy(j_hbm.at[iref], gref, sg)

        def full_body(k, j, guard_first):
            uref, vref, iref, gref, oref, sg, si, so = par_sets[j & 1]
            zref = zsets[j & 3]
            z2ref = zsets[(j + 2) & 3]
            p0 = base + k * C
            p2 = p0 + 2 * C
            hu = pltpu.async_copy(u_hbm.at[pl.ds(p2, C)], uref, si)
            hv = pltpu.async_copy(v_hbm.at[pl.ds(p2, C)], vref, si)
            hz = pltpu.async_copy(z_hbm.at[pl.ds(p2, C)], z2ref, si)
            pltpu.make_async_copy(j_hbm.at[iref], gref, sg).wait()

            def drain_store():
                pltpu.make_async_copy(
                    oref, out_hbm.at[pl.ds(p0 // 128, G)], so).wait()

            if guard_first:
                pl.when(k >= 2)(drain_store)
            else:
                drain_store()
            hu.wait()
            hv.wait()
            hz.wait()

            def fused(gg, pv):
                for b in range(8):
                    model_step(gg, b, zref, gref, oref, pv)
                    idx_step(gg, b, uref, vref, iref)
                return pv

            lax.fori_loop(0, G, fused, load_pats())
            pltpu.async_copy(j_hbm.at[iref], gref, sg)
            pltpu.async_copy(oref, out_hbm.at[pl.ds(p0 // 128, G)], so)

        def tail_body(k, par):
            _, _, iref, gref, oref, sg, _, so = par_sets[par & 1]
            zref = zsets[par & 3]
            pltpu.make_async_copy(j_hbm.at[iref], gref, sg).wait()
            p0 = base + k * C
            pltpu.make_async_copy(
                oref, out_hbm.at[pl.ds(p0 // 128, G)], so).wait()

            def model_only(gg, pv):
                for b in range(8):
                    model_step(gg, b, zref, gref, oref, pv)
                return pv

            lax.fori_loop(0, G, model_only, load_pats())
            pltpu.sync_copy(oref, out_hbm.at[pl.ds(p0 // 128, G)])

        launch(0, 0)
        launch(1, 1)

        def body(p2, carry):
            for j in range(4):
                full_body(4 * p2 + j, j, True)
            return carry

        lax.fori_loop(0, nchunks // 4 - 1, body, 0)
        full_body(nchunks - 4, nchunks - 4, False)
        full_body(nchunks - 3, nchunks - 3, False)
        tail_body(nchunks - 2, nchunks - 2)
        tail_body(nchunks - 1, nchunks - 1)

    return kern(u1, v1, z1, jraw, pats)


def kernel(u, v, z, J, B, beta, gamma):
    n = u.shape[0]
    h, w, _ = J.shape
    th, tw = (h + 7) // 8, (w + 127) // 128
    jraw = (J.transpose(2, 0, 1)
             .reshape(3, th, 8, tw, 128)
             .transpose(0, 1, 3, 2, 4)
             .reshape(3 * th * tw * 8 * 128))
    pats = jnp.repeat(
        jnp.concatenate([-beta, -gamma, B]).astype(jnp.float32)[:, None],
        16, axis=1)
    out4 = _run(u.astype(jnp.int32), v.astype(jnp.int32), z, jraw, pats,
                n=n, h=h, w=w, c_chunk=4096)
    return out4[:, :3, :].transpose(0, 2, 1).reshape(n, 3)
